# baseline (speedup 1.0000x reference)
import functools

import jax
import jax.numpy as jnp
from jax import lax
from jax.experimental import pallas as pl
from jax.experimental.pallas import tpu as pltpu
from jax.experimental.pallas import tpu_sc as plsc


def _logits_body(emb_ref, w_ref, b_ref, m_ref):
    m_ref[...] = (
        lax.dot_general(
            emb_ref[...],
            w_ref[...],
            dimension_numbers=(((1,), (1,)), ((), ())),
            preferred_element_type=jnp.float32,
        )
        + b_ref[...]
    )


def _compute_logits(emb, W, b):
    v, _ = W.shape
    return pl.pallas_call(
        _logits_body,
        out_shape=jax.ShapeDtypeStruct((emb.shape[0], v), jnp.float32),
    )(emb, W, b.reshape(1, v))


@functools.lru_cache(maxsize=None)
def _make_gather(B, V, CH):
    info = plsc.get_sparse_core_info()
    nc, ns = info.num_cores, info.num_subcores
    nw = nc * ns
    assert B % (CH * nw) == 0 and CH % 8 == 0 and CH <= 128
    b_per_w = B // nw
    n_ch = b_per_w // CH
    mesh = plsc.VectorSubcoreMesh(core_axis_name="c", subcore_axis_name="s")

    @functools.partial(
        pl.kernel,
        out_type=jax.ShapeDtypeStruct((B, V), jnp.float32),
        mesh=mesh,
        scratch_types=[
            pltpu.VMEM((b_per_w,), jnp.int32),
            pltpu.VMEM((CH, V), jnp.float32),
            pltpu.SemaphoreType.DMA,
        ],
        compiler_params=pltpu.CompilerParams(use_tc_tiling_on_sc=False),
    )
    def gather(table_hbm, idx_hbm, out_hbm, idx_v, rows_v, sem):
        wid = lax.axis_index("s") * nc + lax.axis_index("c")
        base = wid * b_per_w
        pltpu.sync_copy(idx_hbm.at[pl.ds(base, b_per_w)], idx_v)

        def body(i, carry):
            cb = i * CH
            pltpu.async_copy(
                table_hbm.at[idx_v.at[pl.ds(cb, CH)]], rows_v, sem
            ).wait()
            pltpu.sync_copy(rows_v, out_hbm.at[pl.ds(base + cb, CH)])
            return carry

        lax.fori_loop(0, n_ch, body, 0)

    return gather


def kernel(input_seq, emb_table, W, b):
    bsz, seq = input_seq.shape
    v = W.shape[0]
    m = _compute_logits(emb_table, W, b)
    idx = input_seq.reshape(-1).astype(jnp.int32)
    out = _make_gather(bsz * seq, v, 64)(m, idx)
    return out.reshape(bsz, seq, v)


# speedup vs baseline: 1.0195x; 1.0195x over previous
import functools

import jax
import jax.numpy as jnp
from jax import lax
from jax.experimental import pallas as pl
from jax.experimental.pallas import tpu as pltpu
from jax.experimental.pallas import tpu_sc as plsc


def _logits_body(emb_ref, w_ref, b_ref, m_ref):
    m_ref[...] = (
        lax.dot_general(
            emb_ref[...],
            w_ref[...],
            dimension_numbers=(((1,), (1,)), ((), ())),
            preferred_element_type=jnp.float32,
        )
        + b_ref[...]
    )


def _compute_logits(emb, W, b):
    v, _ = W.shape
    return pl.pallas_call(
        _logits_body,
        out_shape=jax.ShapeDtypeStruct((emb.shape[0], v), jnp.float32),
    )(emb, W, b.reshape(1, v))


@functools.lru_cache(maxsize=None)
def _make_gather(B, V, CH):
    info = plsc.get_sparse_core_info()
    nc, ns = info.num_cores, info.num_subcores
    nw = nc * ns
    assert B % (CH * nw) == 0 and CH % 8 == 0 and CH <= 128
    b_per_w = B // nw
    n_ch = b_per_w // CH
    assert n_ch % 2 == 0
    mesh = plsc.VectorSubcoreMesh(core_axis_name="c", subcore_axis_name="s")

    @functools.partial(
        pl.kernel,
        out_type=jax.ShapeDtypeStruct((B, V), jnp.float32),
        mesh=mesh,
        scratch_types=[
            pltpu.VMEM((b_per_w,), jnp.int32),
            pltpu.VMEM((CH, V), jnp.float32),
            pltpu.VMEM((CH, V), jnp.float32),
            pltpu.SemaphoreType.DMA,
            pltpu.SemaphoreType.DMA,
        ],
        compiler_params=pltpu.CompilerParams(use_tc_tiling_on_sc=False),
    )
    def gather(table_hbm, idx_hbm, out_hbm, idx_v, rows0, rows1, sem0, sem1):
        wid = lax.axis_index("s") * nc + lax.axis_index("c")
        base = wid * b_per_w
        pltpu.sync_copy(idx_hbm.at[pl.ds(base, b_per_w)], idx_v)

        bufs = ((rows0, sem0), (rows1, sem1))

        def start(i, buf, sem):
            pltpu.async_copy(
                table_hbm.at[idx_v.at[pl.ds(i * CH, CH)]], buf, sem
            )

        def finish(i, buf, sem):
            pltpu.make_async_copy(
                table_hbm.at[idx_v.at[pl.ds(i * CH, CH)]], buf, sem
            ).wait()
            pltpu.sync_copy(buf, out_hbm.at[pl.ds(base + i * CH, CH)])

        start(0, *bufs[0])

        def body(j, carry):
            i0 = j * 2
            start(i0 + 1, *bufs[1])
            finish(i0, *bufs[0])

            @pl.when(j < n_ch // 2 - 1)
            def _():
                start(i0 + 2, *bufs[0])

            finish(i0 + 1, *bufs[1])
            return carry

        lax.fori_loop(0, n_ch // 2, body, 0)

    return gather


def kernel(input_seq, emb_table, W, b):
    bsz, seq = input_seq.shape
    v = W.shape[0]
    m = _compute_logits(emb_table, W, b)
    idx = input_seq.reshape(-1).astype(jnp.int32)
    out = _make_gather(bsz * seq, v, 32)(m, idx)
    return out.reshape(bsz, seq, v)


# speedup vs baseline: 1.4419x; 1.4144x over previous
import functools

import jax
import jax.numpy as jnp
from jax import lax
from jax.experimental import pallas as pl
from jax.experimental.pallas import tpu as pltpu
from jax.experimental.pallas import tpu_sc as plsc


@functools.lru_cache(maxsize=None)
def _make_gather(B, D, CH):
    info = plsc.get_sparse_core_info()
    nc, ns = info.num_cores, info.num_subcores
    nw = nc * ns
    b_per_w = B // nw
    n_ch = b_per_w // CH
    assert B % (CH * nw) == 0 and CH % 8 == 0 and CH <= 128
    assert n_ch % 2 == 0
    mesh = plsc.VectorSubcoreMesh(core_axis_name="c", subcore_axis_name="s")

    @functools.partial(
        pl.kernel,
        out_type=jax.ShapeDtypeStruct((B, D), jnp.float32),
        mesh=mesh,
        scratch_types=[
            pltpu.VMEM((b_per_w,), jnp.int32),
            pltpu.VMEM((CH, D), jnp.float32),
            pltpu.VMEM((CH, D), jnp.float32),
            pltpu.SemaphoreType.DMA,
            pltpu.SemaphoreType.DMA,
        ],
    )
    def gather(table_hbm, idx_hbm, out_hbm, idx_v, rows0, rows1, sem0, sem1):
        wid = lax.axis_index("s") * nc + lax.axis_index("c")
        base = wid * b_per_w
        pltpu.sync_copy(idx_hbm.at[pl.ds(base, b_per_w)], idx_v)

        bufs = ((rows0, sem0), (rows1, sem1))

        def start(i, buf, sem):
            pltpu.async_copy(
                table_hbm.at[idx_v.at[pl.ds(i * CH, CH)]], buf, sem
            )

        def finish(i, buf, sem):
            pltpu.make_async_copy(
                table_hbm.at[idx_v.at[pl.ds(i * CH, CH)]], buf, sem
            ).wait()
            pltpu.sync_copy(buf, out_hbm.at[pl.ds(base + i * CH, CH)])

        start(0, *bufs[0])

        def body(j, carry):
            i0 = j * 2
            start(i0 + 1, *bufs[1])
            finish(i0, *bufs[0])

            @pl.when(j < n_ch // 2 - 1)
            def _():
                start(i0 + 2, *bufs[0])

            finish(i0 + 1, *bufs[1])
            return carry

        lax.fori_loop(0, n_ch // 2, body, 0)

    return gather


def _proj_body(e_ref, w_ref, b_ref, o_ref):
    o_ref[...] = (
        lax.dot_general(
            e_ref[...],
            w_ref[...],
            dimension_numbers=(((1,), (1,)), ((), ())),
            preferred_element_type=jnp.float32,
        )
        + b_ref[...]
    )


def _project(E, W, b, BM):
    B, D = E.shape
    V = W.shape[0]
    return pl.pallas_call(
        _proj_body,
        grid=(B // BM,),
        in_specs=[
            pl.BlockSpec((BM, D), lambda i: (i, 0)),
            pl.BlockSpec((V, D), lambda i: (0, 0)),
            pl.BlockSpec((1, V), lambda i: (0, 0)),
        ],
        out_specs=pl.BlockSpec((BM, V), lambda i: (i, 0)),
        out_shape=jax.ShapeDtypeStruct((B, V), jnp.float32),
    )(E, W, b.reshape(1, V))


def kernel(input_seq, emb_table, W, b):
    bsz, seq = input_seq.shape
    v = W.shape[0]
    idx = input_seq.reshape(-1).astype(jnp.int32)
    E = _make_gather(bsz * seq, emb_table.shape[1], 80)(emb_table, idx)
    out = _project(E, W, b, 512)
    return out.reshape(bsz, seq, v)


# speedup vs baseline: 1.8597x; 1.2897x over previous
import functools

import jax
import jax.numpy as jnp
from jax import lax
from jax.experimental import pallas as pl
from jax.experimental.pallas import tpu as pltpu
from jax.experimental.pallas import tpu_sc as plsc


@functools.lru_cache(maxsize=None)
def _make_gather(B, D, CH):
    info = plsc.get_sparse_core_info()
    nc, ns = info.num_cores, info.num_subcores
    nw = nc * ns
    b_per_w = B // nw
    n_ch = b_per_w // CH
    assert B % (CH * nw) == 0 and CH % 8 == 0 and CH <= 128
    assert n_ch % 2 == 0
    mesh = plsc.VectorSubcoreMesh(core_axis_name="c", subcore_axis_name="s")

    @functools.partial(
        pl.kernel,
        out_type=jax.ShapeDtypeStruct((B, D), jnp.float32),
        mesh=mesh,
        scratch_types=[
            pltpu.VMEM((b_per_w,), jnp.int32),
            pltpu.VMEM((CH, D), jnp.float32),
            pltpu.VMEM((CH, D), jnp.float32),
            pltpu.SemaphoreType.DMA,
            pltpu.SemaphoreType.DMA,
        ],
    )
    def gather(table_hbm, idx_hbm, out_hbm, idx_v, rows0, rows1, sem0, sem1):
        wid = lax.axis_index("s") * nc + lax.axis_index("c")
        base = wid * b_per_w
        pltpu.sync_copy(idx_hbm.at[pl.ds(base, b_per_w)], idx_v)

        bufs = ((rows0, sem0), (rows1, sem1))

        def start(i, buf, sem):
            pltpu.async_copy(
                table_hbm.at[idx_v.at[pl.ds(i * CH, CH)]], buf, sem
            )

        def finish(i, buf, sem):
            pltpu.make_async_copy(
                table_hbm.at[idx_v.at[pl.ds(i * CH, CH)]], buf, sem
            ).wait()
            pltpu.sync_copy(buf, out_hbm.at[pl.ds(base + i * CH, CH)])

        start(0, *bufs[0])

        def body(j, carry):
            i0 = j * 2
            start(i0 + 1, *bufs[1])
            finish(i0, *bufs[0])

            @pl.when(j < n_ch // 2 - 1)
            def _():
                start(i0 + 2, *bufs[0])

            finish(i0 + 1, *bufs[1])
            return carry

        lax.fori_loop(0, n_ch // 2, body, 0)

    return gather


def _proj_body(bs, e_ref, w_ref, b_ref, o_ref):
    res = (
        lax.dot_general(
            e_ref[...],
            w_ref[...],
            dimension_numbers=(((1,), (1,)), ((), ())),
            preferred_element_type=jnp.float32,
        )
        + b_ref[...]
    )
    o_ref[...] = res.reshape(o_ref.shape)


def _project(E, W, b, bsz, seq, BB):
    B, D = E.shape
    V = W.shape[0]
    return pl.pallas_call(
        functools.partial(_proj_body, seq),
        grid=(bsz // BB,),
        in_specs=[
            pl.BlockSpec((BB * seq, D), lambda i: (i, 0)),
            pl.BlockSpec((V, D), lambda i: (0, 0)),
            pl.BlockSpec((1, V), lambda i: (0, 0)),
        ],
        out_specs=pl.BlockSpec((BB, seq, V), lambda i: (i, 0, 0)),
        out_shape=jax.ShapeDtypeStruct((bsz, seq, V), jnp.float32),
    )(E, W, b.reshape(1, V))


def kernel(input_seq, emb_table, W, b):
    bsz, seq = input_seq.shape
    v = W.shape[0]
    idx = input_seq.reshape(-1).astype(jnp.int32)
    E = _make_gather(bsz * seq, emb_table.shape[1], 80)(emb_table, idx)
    return _project(E, W, b, bsz, seq, 16)
